# Initial kernel scaffold; baseline (speedup 1.0000x reference)
import jax
import jax.numpy as jnp
from jax.experimental import pallas as pl


def kernel(xyz_s, xyz_t, feature_s, feature_t, W, gamma, beta):
    raise NotImplementedError("write your pallas kernel here")


# speedup vs baseline: 9.7351x; 9.7351x over previous
import functools

import jax
import jax.numpy as jnp
from jax import lax
from jax.experimental import pallas as pl
from jax.experimental.pallas import tpu as pltpu
from jax.experimental.pallas import tpu_sc as plsc

KNN = 16
GW = 16
RB = 256


def _scores_body(fsb_ref, ftf_ref, fsf_ref, ftb_ref, xyzsb_ref, a_ref, w3_ref,
                 s_ref, gmax_ref, g_ref):
    fsb = fsb_ref[0]
    ftf = ftf_ref[0]
    d = lax.dot_general(fsb, ftf, (((0,), (0,)), ((), ())))
    xx = jnp.sum(fsf_ref[0] * fsf_ref[0], axis=0)
    yy = jnp.sum(ftb_ref[0] * ftb_ref[0], axis=0)
    s = (2.0 * d - xx[None, :]) - yy[:, None]
    s_ref[0] = s
    m = s
    while m.shape[1] > s.shape[1] // GW:
        h = m.shape[1] // 2
        m = jnp.maximum(m[:, :h], m[:, h:])
    gmax_ref[0] = m
    g = lax.dot_general(fsb, a_ref[...], (((0,), (1,)), ((), ())))
    g += lax.dot_general(xyzsb_ref[0], w3_ref[...], (((1,), (1,)), ((), ())))
    g_ref[0] = g


def _q_block(ftb, xyztb, w1n, w43):
    q = lax.dot_general(ftb, w1n, (((0,), (1,)), ((), ())))
    q += lax.dot_general(xyztb, w43, (((1,), (1,)), ((), ())))
    return q


def _stats_body(ssum_ref, ssq_ref, ftb_ref, xyztb_ref, w1n_ref, w43_ref,
                sumh_ref, sqh_ref):
    b = pl.program_id(0)
    i = pl.program_id(1)

    @pl.when((b == 0) & (i == 0))
    def _init():
        sumh_ref[...] = jnp.zeros_like(sumh_ref)
        sqh_ref[...] = jnp.zeros_like(sqh_ref)

    q = _q_block(ftb_ref[0], xyztb_ref[0], w1n_ref[...], w43_ref[...])
    ssum = ssum_ref[0]
    ssq = ssq_ref[0]
    kf = float(KNN)
    ph = jnp.sum(ssum + kf * q, axis=0)
    pq = jnp.sum(ssq + 2.0 * q * ssum + kf * q * q, axis=0)
    sumh_ref[...] += jnp.broadcast_to(ph[None, :], sumh_ref.shape)
    sqh_ref[...] += jnp.broadcast_to(pq[None, :], sqh_ref.shape)


def _final_body(smax_ref, smin_ref, ftb_ref, xyztb_ref, w1n_ref, w43_ref,
                gamma_ref, beta_ref, sumh_ref, sqh_ref, o_ref, *, m_total):
    q = _q_block(ftb_ref[0], xyztb_ref[0], w1n_ref[...], w43_ref[...])
    mean = sumh_ref[0:1, :] / m_total
    var = sqh_ref[0:1, :] / m_total - mean * mean
    inv = lax.rsqrt(var + 1e-5)
    scale = gamma_ref[...] * inv
    hmax = smax_ref[0] + q
    hmin = smin_ref[0] + q
    hsel = jnp.where(scale >= 0.0, hmax, hmin)
    t = (hsel - mean) * scale + beta_ref[...]
    t = jnp.where(t >= 0.0, t, 0.2 * t)
    o_ref[0] = t.T


def _make_sc_topk(B, N, C, NG):
    info = plsc.get_sparse_core_info()
    nc, ns = info.num_cores, info.num_subcores
    nw = nc * ns
    rows = B * N
    rpw = rows // nw
    mesh = plsc.VectorSubcoreMesh(core_axis_name="c", subcore_axis_name="s")
    sds = jax.ShapeDtypeStruct((rows, C), jnp.float32)

    @functools.partial(
        pl.kernel,
        out_type=(sds, sds, sds, sds),
        mesh=mesh,
        scratch_types=[
            pltpu.VMEM((NG,), jnp.float32),
            pltpu.VMEM((2, 128), jnp.int32),
            pltpu.VMEM((KNN * GW,), jnp.float32),
            pltpu.VMEM((KNN, C), jnp.float32),
            pltpu.VMEM((C,), jnp.float32),
            pltpu.VMEM((C,), jnp.float32),
            pltpu.VMEM((C,), jnp.float32),
            pltpu.VMEM((C,), jnp.float32),
            pltpu.SemaphoreType.DMA,
            pltpu.SemaphoreType.DMA,
        ],
        compiler_params=pltpu.CompilerParams(needs_layout_passes=False,
                                             use_tc_tiling_on_sc=False),
    )
    def sc_topk(gmax_hbm, scores_hbm, g_hbm,
                smax_hbm, smin_hbm, ssum_hbm, ssq_hbm,
                grow, cidx, cand, gbuf, omx, omn, osm, osq, sem1, sem2):
        wid = lax.axis_index("s") * nc + lax.axis_index("c")
        base = wid * rpw

        def merge(a, b):
            av, ai = a
            bv = lax.rev(b[0], (0,))
            bi = lax.rev(b[1], (0,))
            m = av >= bv
            mv = jnp.where(m, av, bv)
            mi = jnp.where(m, ai, bi)
            return plsc.sort_key_val(mv, mi, descending=True)

        def tourney(chunks):
            pairs = [plsc.sort_key_val(v, i, descending=True) for v, i in chunks]
            while len(pairs) > 1:
                pairs = [merge(pairs[j], pairs[j + 1])
                         for j in range(0, len(pairs), 2)]
            return pairs[0]

        def body(r, _):
            row = base + r
            pltpu.sync_copy(gmax_hbm.at[row], grow)
            iota = lax.iota(jnp.int32, GW)
            _, gid = tourney([(grow[pl.ds(c * GW, GW)], iota + c * GW)
                              for c in range(NG // GW)])
            rowflat = row * N
            tc = [gid[c] for c in range(KNN)]
            for c in range(KNN):
                cidx[c // 8, pl.ds((c % 8) * GW, GW)] = (
                    rowflat + tc[c] + NG * iota)
            d1 = pltpu.async_copy(scores_hbm.at[cidx.at[0]],
                                  cand.at[pl.ds(0, 128)], sem1)
            d2 = pltpu.async_copy(scores_hbm.at[cidx.at[1]],
                                  cand.at[pl.ds(128, 128)], sem1)
            d1.wait()
            d2.wait()
            _, jid = tourney([(cand[pl.ds(c * GW, GW)], tc[c] + NG * iota)
                              for c in range(KNN)])
            rowbase = (row // N) * N
            pltpu.async_copy(g_hbm.at[jid + rowbase], gbuf, sem2).wait()
            for ch in range(C // GW):
                sl = pl.ds(ch * GW, GW)
                v = gbuf[0, sl]
                mx = v
                mn = v
                s1 = v
                s2 = v * v
                for rr in range(1, KNN):
                    v = gbuf[rr, sl]
                    mx = jnp.maximum(mx, v)
                    mn = jnp.minimum(mn, v)
                    s1 = s1 + v
                    s2 = s2 + v * v
                omx[sl] = mx
                omn[sl] = mn
                osm[sl] = s1
                osq[sl] = s2
            pltpu.sync_copy(omx, smax_hbm.at[row])
            pltpu.sync_copy(omn, smin_hbm.at[row])
            pltpu.sync_copy(osm, ssum_hbm.at[row])
            pltpu.sync_copy(osq, ssq_hbm.at[row])
            return ()

        lax.fori_loop(0, rpw, body, ())

    return sc_topk


def kernel(xyz_s, xyz_t, feature_s, feature_t, W, gamma, beta):
    B, C, N = feature_s.shape
    NG = N // GW
    NB = N // RB
    f32 = jnp.float32

    W1 = W[:, :C]
    W2 = W[:, C:2 * C]
    W3 = W[:, 2 * C:2 * C + 3]
    W4 = W[:, 2 * C + 3:]
    A = W1 + W2
    W1n = -W1
    W43 = W4 - W3
    gamma2 = gamma.reshape(1, C)
    beta2 = beta.reshape(1, C)

    blk_cr = pl.BlockSpec((1, C, RB), lambda b, i: (b, 0, i))
    blk_cn = pl.BlockSpec((1, C, N), lambda b, i: (b, 0, 0))
    blk_r3 = pl.BlockSpec((1, RB, 3), lambda b, i: (b, i, 0))
    blk_r64 = pl.BlockSpec((1, RB, C), lambda b, i: (b, i, 0))
    full2 = lambda shape: pl.BlockSpec(shape, lambda b, i: (0, 0))

    scores, gmax, g = pl.pallas_call(
        _scores_body,
        grid=(B, NB),
        in_specs=[blk_cr, blk_cn, blk_cn, blk_cr, blk_r3,
                  full2((C, C)), full2((C, 3))],
        out_specs=[pl.BlockSpec((1, RB, N), lambda b, i: (b, i, 0)),
                   pl.BlockSpec((1, RB, NG), lambda b, i: (b, i, 0)),
                   blk_r64],
        out_shape=[jax.ShapeDtypeStruct((B, N, N), f32),
                   jax.ShapeDtypeStruct((B, N, NG), f32),
                   jax.ShapeDtypeStruct((B, N, C), f32)],
    )(feature_s, feature_t, feature_s, feature_t, xyz_s, A, W3)

    sc_topk = _make_sc_topk(B, N, C, NG)
    smax, smin, ssum, ssq = sc_topk(
        gmax.reshape(B * N, NG),
        scores.reshape(B * N * N),
        g.reshape(B * N, C))
    smax = smax.reshape(B, N, C)
    smin = smin.reshape(B, N, C)
    ssum = ssum.reshape(B, N, C)
    ssq = ssq.reshape(B, N, C)

    stat_spec = pl.BlockSpec((8, C), lambda b, i: (0, 0))
    sumh, sqh = pl.pallas_call(
        _stats_body,
        grid=(B, NB),
        in_specs=[blk_r64, blk_r64, blk_cr, blk_r3,
                  full2((C, C)), full2((C, 3))],
        out_specs=[stat_spec, stat_spec],
        out_shape=[jax.ShapeDtypeStruct((8, C), f32),
                   jax.ShapeDtypeStruct((8, C), f32)],
        compiler_params=pltpu.CompilerParams(
            dimension_semantics=("arbitrary", "arbitrary")),
    )(ssum, ssq, feature_t, xyz_t, W1n, W43)

    m_total = float(B * N * KNN)
    out = pl.pallas_call(
        functools.partial(_final_body, m_total=m_total),
        grid=(B, NB),
        in_specs=[blk_r64, blk_r64, blk_cr, blk_r3,
                  full2((C, C)), full2((C, 3)),
                  full2((1, C)), full2((1, C)),
                  stat_spec, stat_spec],
        out_specs=pl.BlockSpec((1, C, RB), lambda b, i: (b, 0, i)),
        out_shape=jax.ShapeDtypeStruct((B, C, N), f32),
    )(smax, smin, feature_t, xyz_t, W1n, W43, gamma2, beta2, sumh, sqh)
    return out


# speedup vs baseline: 13.9203x; 1.4299x over previous
import functools

import jax
import jax.numpy as jnp
from jax import lax
from jax.experimental import pallas as pl
from jax.experimental.pallas import tpu as pltpu
from jax.experimental.pallas import tpu_sc as plsc

KNN = 16
GW = 16
RB = 256


def _scores_body(fsb_ref, ftf_ref, fsf_ref, ftb_ref, xyzsb_ref, a_ref, w3_ref,
                 s_ref, gmax_ref, g_ref):
    fsb = fsb_ref[0]
    ftf = ftf_ref[0]
    d = lax.dot_general(fsb, ftf, (((0,), (0,)), ((), ())))
    xx = jnp.sum(fsf_ref[0] * fsf_ref[0], axis=0)
    yy = jnp.sum(ftb_ref[0] * ftb_ref[0], axis=0)
    s = (2.0 * d - xx[None, :]) - yy[:, None]
    s_ref[0] = s
    m = s
    while m.shape[1] > s.shape[1] // GW:
        h = m.shape[1] // 2
        m = jnp.maximum(m[:, :h], m[:, h:])
    gmax_ref[0] = m
    g = lax.dot_general(fsb, a_ref[...], (((0,), (1,)), ((), ())))
    g += lax.dot_general(xyzsb_ref[0], w3_ref[...], (((1,), (1,)), ((), ())))
    g_ref[0] = g


def _q_block(ftb, xyztb, w1n, w43):
    q = lax.dot_general(ftb, w1n, (((0,), (1,)), ((), ())))
    q += lax.dot_general(xyztb, w43, (((1,), (1,)), ((), ())))
    return q


def _stats_body(out4_ref, ftb_ref, xyztb_ref, w1n_ref, w43_ref,
                sumh_ref, sqh_ref):
    b = pl.program_id(0)
    i = pl.program_id(1)

    @pl.when((b == 0) & (i == 0))
    def _init():
        sumh_ref[...] = jnp.zeros_like(sumh_ref)
        sqh_ref[...] = jnp.zeros_like(sqh_ref)

    q = _q_block(ftb_ref[0], xyztb_ref[0], w1n_ref[...], w43_ref[...])
    nch = out4_ref.shape[2] // 4
    ssum = out4_ref[0, :, 2 * nch:3 * nch]
    ssq = out4_ref[0, :, 3 * nch:]
    kf = float(KNN)
    ph = jnp.sum(ssum + kf * q, axis=0)
    pq = jnp.sum(ssq + 2.0 * q * ssum + kf * q * q, axis=0)
    sumh_ref[...] += jnp.broadcast_to(ph[None, :], sumh_ref.shape)
    sqh_ref[...] += jnp.broadcast_to(pq[None, :], sqh_ref.shape)


def _final_body(out4_ref, ftb_ref, xyztb_ref, w1n_ref, w43_ref,
                gamma_ref, beta_ref, sumh_ref, sqh_ref, o_ref, *, m_total):
    q = _q_block(ftb_ref[0], xyztb_ref[0], w1n_ref[...], w43_ref[...])
    mean = sumh_ref[0:1, :] / m_total
    var = sqh_ref[0:1, :] / m_total - mean * mean
    inv = lax.rsqrt(var + 1e-5)
    scale = gamma_ref[...] * inv
    nch = out4_ref.shape[2] // 4
    hmax = out4_ref[0, :, :nch] + q
    hmin = out4_ref[0, :, nch:2 * nch] + q
    hsel = jnp.where(scale >= 0.0, hmax, hmin)
    t = (hsel - mean) * scale + beta_ref[...]
    t = jnp.where(t >= 0.0, t, 0.2 * t)
    o_ref[0] = t.T


def _make_sc_topk(B, N, C, NG):
    info = plsc.get_sparse_core_info()
    nc, ns = info.num_cores, info.num_subcores
    nw = nc * ns
    rows = B * N
    rpw = rows // nw
    mesh = plsc.VectorSubcoreMesh(core_axis_name="c", subcore_axis_name="s")

    @functools.partial(
        pl.kernel,
        out_type=jax.ShapeDtypeStruct((rows, 4 * C), jnp.float32),
        mesh=mesh,
        scratch_types=[
            pltpu.VMEM((2, NG), jnp.float32),
            pltpu.VMEM((4, 128), jnp.int32),
            pltpu.VMEM((2 * KNN * GW,), jnp.float32),
            pltpu.VMEM((2, KNN, C), jnp.float32),
            pltpu.VMEM((2, 4 * C), jnp.float32),
            pltpu.SemaphoreType.DMA,
            pltpu.SemaphoreType.DMA,
            pltpu.SemaphoreType.DMA,
        ],
        compiler_params=pltpu.CompilerParams(needs_layout_passes=False,
                                             use_tc_tiling_on_sc=False),
    )
    def sc_topk(gmax_hbm, scores_hbm, g_hbm, out_hbm,
                grow, cidx, cand, gbuf, orow, semg, semc, seme):
        wid = lax.axis_index("s") * nc + lax.axis_index("c")
        base = wid * rpw
        iota = lax.iota(jnp.int32, GW)

        def merge(a, b):
            av, ai = a
            bv = lax.rev(b[0], (0,))
            bi = lax.rev(b[1], (0,))
            m = av >= bv
            mv = jnp.where(m, av, bv)
            mi = jnp.where(m, ai, bi)
            return plsc.sort_key_val(mv, mi, descending=True)

        def tourney(chunks):
            pairs = [plsc.sort_key_val(v, i, descending=True) for v, i in chunks]
            while len(pairs) > 1:
                pairs = [merge(pairs[j], pairs[j + 1])
                         for j in range(0, len(pairs), 2)]
            return pairs[0]

        def stage1(s):
            _, gid = tourney([(grow[s, pl.ds(c * GW, GW)], iota + c * GW)
                              for c in range(NG // GW)])
            return [gid[c] for c in range(KNN)]

        def fire_cand(s, row, tc):
            rowflat = row * N
            for c in range(KNN):
                cidx[2 * s + c // 8, pl.ds((c % 8) * GW, GW)] = (
                    rowflat + tc[c] + NG * iota)
            return [
                pltpu.async_copy(scores_hbm.at[cidx.at[2 * s + j]],
                                 cand.at[pl.ds((2 * s + j) * 128, 128)], semc)
                for j in range(2)]

        def stage2(s, row, tc):
            _, jid = tourney([(cand[pl.ds((s * KNN + c) * GW, GW)],
                               tc[c] + NG * iota) for c in range(KNN)])
            rowbase = (row // N) * N
            return pltpu.async_copy(g_hbm.at[jid + rowbase], gbuf.at[s], seme)

        def reduce_store(s, row):
            for ch in range(C // GW):
                sl0 = ch * GW
                v = gbuf[s, 0, pl.ds(sl0, GW)]
                mx = v
                mn = v
                s1 = v
                s2 = v * v
                for rr in range(1, KNN):
                    v = gbuf[s, rr, pl.ds(sl0, GW)]
                    mx = jnp.maximum(mx, v)
                    mn = jnp.minimum(mn, v)
                    s1 = s1 + v
                    s2 = s2 + v * v
                orow[s, pl.ds(sl0, GW)] = mx
                orow[s, pl.ds(C + sl0, GW)] = mn
                orow[s, pl.ds(2 * C + sl0, GW)] = s1
                orow[s, pl.ds(3 * C + sl0, GW)] = s2
            return pltpu.async_copy(orow.at[s], out_hbm.at[row], semg)

        def body(r, _):
            row0 = base + 2 * r
            row1 = row0 + 1
            d0 = pltpu.async_copy(gmax_hbm.at[row0], grow.at[0], semg)
            d1 = pltpu.async_copy(gmax_hbm.at[row1], grow.at[1], semg)
            d0.wait()
            d1.wait()
            tc0 = stage1(0)
            c0 = fire_cand(0, row0, tc0)
            tc1 = stage1(1)
            c1 = fire_cand(1, row1, tc1)
            c0[0].wait()
            c0[1].wait()
            c1[0].wait()
            c1[1].wait()
            e0 = stage2(0, row0, tc0)
            e1 = stage2(1, row1, tc1)
            e0.wait()
            e1.wait()
            o0 = reduce_store(0, row0)
            o1 = reduce_store(1, row1)
            o0.wait()
            o1.wait()
            return ()

        lax.fori_loop(0, rpw // 2, body, ())

    return sc_topk


def kernel(xyz_s, xyz_t, feature_s, feature_t, W, gamma, beta):
    B, C, N = feature_s.shape
    NG = N // GW
    NB = N // RB
    f32 = jnp.float32

    W1 = W[:, :C]
    W2 = W[:, C:2 * C]
    W3 = W[:, 2 * C:2 * C + 3]
    W4 = W[:, 2 * C + 3:]
    A = W1 + W2
    W1n = -W1
    W43 = W4 - W3
    gamma2 = gamma.reshape(1, C)
    beta2 = beta.reshape(1, C)

    blk_cr = pl.BlockSpec((1, C, RB), lambda b, i: (b, 0, i))
    blk_cn = pl.BlockSpec((1, C, N), lambda b, i: (b, 0, 0))
    blk_r3 = pl.BlockSpec((1, RB, 3), lambda b, i: (b, i, 0))
    blk_r64 = pl.BlockSpec((1, RB, C), lambda b, i: (b, i, 0))
    full2 = lambda shape: pl.BlockSpec(shape, lambda b, i: (0, 0))

    scores, gmax, g = pl.pallas_call(
        _scores_body,
        grid=(B, NB),
        in_specs=[blk_cr, blk_cn, blk_cn, blk_cr, blk_r3,
                  full2((C, C)), full2((C, 3))],
        out_specs=[pl.BlockSpec((1, RB, N), lambda b, i: (b, i, 0)),
                   pl.BlockSpec((1, RB, NG), lambda b, i: (b, i, 0)),
                   blk_r64],
        out_shape=[jax.ShapeDtypeStruct((B, N, N), f32),
                   jax.ShapeDtypeStruct((B, N, NG), f32),
                   jax.ShapeDtypeStruct((B, N, C), f32)],
    )(feature_s, feature_t, feature_s, feature_t, xyz_s, A, W3)

    sc_topk = _make_sc_topk(B, N, C, NG)
    out4 = sc_topk(
        gmax.reshape(B * N, NG),
        scores.reshape(B * N * N),
        g.reshape(B * N, C)).reshape(B, N, 4 * C)

    blk_o4 = pl.BlockSpec((1, RB, 4 * C), lambda b, i: (b, i, 0))
    stat_spec = pl.BlockSpec((8, C), lambda b, i: (0, 0))
    sumh, sqh = pl.pallas_call(
        _stats_body,
        grid=(B, NB),
        in_specs=[blk_o4, blk_cr, blk_r3,
                  full2((C, C)), full2((C, 3))],
        out_specs=[stat_spec, stat_spec],
        out_shape=[jax.ShapeDtypeStruct((8, C), f32),
                   jax.ShapeDtypeStruct((8, C), f32)],
        compiler_params=pltpu.CompilerParams(
            dimension_semantics=("arbitrary", "arbitrary")),
    )(out4, feature_t, xyz_t, W1n, W43)

    m_total = float(B * N * KNN)
    out = pl.pallas_call(
        functools.partial(_final_body, m_total=m_total),
        grid=(B, NB),
        in_specs=[blk_o4, blk_cr, blk_r3,
                  full2((C, C)), full2((C, 3)),
                  full2((1, C)), full2((1, C)),
                  stat_spec, stat_spec],
        out_specs=pl.BlockSpec((1, C, RB), lambda b, i: (b, 0, i)),
        out_shape=jax.ShapeDtypeStruct((B, C, N), f32),
    )(out4, feature_t, xyz_t, W1n, W43, gamma2, beta2, sumh, sqh)
    return out


# speedup vs baseline: 19.0116x; 1.3657x over previous
import functools

import jax
import jax.numpy as jnp
from jax import lax
from jax.experimental import pallas as pl
from jax.experimental.pallas import tpu as pltpu
from jax.experimental.pallas import tpu_sc as plsc

KNN = 16
GW = 16
RB = 256


def _scores_body(fsb_ref, ftf_ref, fsf_ref, ftb_ref, xyzsb_ref, a_ref, w3_ref,
                 s_ref, gmax_ref, g_ref):
    fsb = fsb_ref[0]
    ftf = ftf_ref[0]
    d = lax.dot_general(fsb, ftf, (((0,), (0,)), ((), ())))
    xx = jnp.sum(fsf_ref[0] * fsf_ref[0], axis=0)
    yy = jnp.sum(ftb_ref[0] * ftb_ref[0], axis=0)
    s = (2.0 * d - xx[None, :]) - yy[:, None]
    s_ref[0] = s
    m = s
    while m.shape[1] > s.shape[1] // GW:
        h = m.shape[1] // 2
        m = jnp.maximum(m[:, :h], m[:, h:])
    gmax_ref[0] = m
    g = lax.dot_general(fsb, a_ref[...], (((0,), (1,)), ((), ())))
    g += lax.dot_general(xyzsb_ref[0], w3_ref[...], (((1,), (1,)), ((), ())))
    g_ref[0] = g


def _q_block(ftb, xyztb, w1n, w43):
    q = lax.dot_general(ftb, w1n, (((0,), (1,)), ((), ())))
    q += lax.dot_general(xyztb, w43, (((1,), (1,)), ((), ())))
    return q


def _stats_body(out4_ref, ftb_ref, xyztb_ref, w1n_ref, w43_ref,
                sumh_ref, sqh_ref):
    b = pl.program_id(0)
    i = pl.program_id(1)

    @pl.when((b == 0) & (i == 0))
    def _init():
        sumh_ref[...] = jnp.zeros_like(sumh_ref)
        sqh_ref[...] = jnp.zeros_like(sqh_ref)

    q = _q_block(ftb_ref[0], xyztb_ref[0], w1n_ref[...], w43_ref[...])
    nch = out4_ref.shape[2] // 4
    ssum = out4_ref[0, :, 2 * nch:3 * nch]
    ssq = out4_ref[0, :, 3 * nch:]
    kf = float(KNN)
    ph = jnp.sum(ssum + kf * q, axis=0)
    pq = jnp.sum(ssq + 2.0 * q * ssum + kf * q * q, axis=0)
    sumh_ref[...] += jnp.broadcast_to(ph[None, :], sumh_ref.shape)
    sqh_ref[...] += jnp.broadcast_to(pq[None, :], sqh_ref.shape)


def _final_body(out4_ref, ftb_ref, xyztb_ref, w1n_ref, w43_ref,
                gamma_ref, beta_ref, sumh_ref, sqh_ref, o_ref, *, m_total):
    q = _q_block(ftb_ref[0], xyztb_ref[0], w1n_ref[...], w43_ref[...])
    mean = sumh_ref[0:1, :] / m_total
    var = sqh_ref[0:1, :] / m_total - mean * mean
    inv = lax.rsqrt(var + 1e-5)
    scale = gamma_ref[...] * inv
    nch = out4_ref.shape[2] // 4
    hmax = out4_ref[0, :, :nch] + q
    hmin = out4_ref[0, :, nch:2 * nch] + q
    hsel = jnp.where(scale >= 0.0, hmax, hmin)
    t = (hsel - mean) * scale + beta_ref[...]
    t = jnp.where(t >= 0.0, t, 0.2 * t)
    o_ref[0] = t.T


def _make_sc_topk(B, N, C, NG):
    info = plsc.get_sparse_core_info()
    nc, ns = info.num_cores, info.num_subcores
    nw = nc * ns
    rows = B * N
    rpw = rows // nw
    mesh = plsc.VectorSubcoreMesh(core_axis_name="c", subcore_axis_name="s")

    @functools.partial(
        pl.kernel,
        out_type=jax.ShapeDtypeStruct((rows, 4 * C), jnp.float32),
        mesh=mesh,
        scratch_types=[
            pltpu.VMEM((2, 2, NG), jnp.float32),
            pltpu.VMEM((2, 4, 128), jnp.int32),
            pltpu.VMEM((2, 2 * KNN * GW), jnp.float32),
            pltpu.VMEM((2, KNN, C), jnp.float32),
            pltpu.VMEM((2, 4 * C), jnp.float32),
            pltpu.SemaphoreType.DMA,
            pltpu.SemaphoreType.DMA,
            pltpu.SemaphoreType.DMA,
            pltpu.SemaphoreType.DMA,
        ],
        compiler_params=pltpu.CompilerParams(needs_layout_passes=False,
                                             use_tc_tiling_on_sc=False),
    )
    def sc_topk(gmax_hbm, scores_hbm, g_hbm, out_hbm,
                grow, cidx, cand, gbuf, orow, semg, semc, seme, semo):
        wid = lax.axis_index("s") * nc + lax.axis_index("c")
        base = wid * rpw
        npairs = rpw // 2
        iota = lax.iota(jnp.int32, GW)

        def merge(a, b):
            av, ai = a
            bv = lax.rev(b[0], (0,))
            bi = lax.rev(b[1], (0,))
            m = av >= bv
            mv = jnp.where(m, av, bv)
            mi = jnp.where(m, ai, bi)
            return plsc.sort_key_val(mv, mi, descending=True)

        def tourney(chunks):
            pairs = [plsc.sort_key_val(v, i, descending=True) for v, i in chunks]
            while len(pairs) > 1:
                pairs = [merge(pairs[j], pairs[j + 1])
                         for j in range(0, len(pairs), 2)]
            return pairs[0]

        def fire_gmax(p, par):
            r0 = base + 2 * p
            pltpu.async_copy(gmax_hbm.at[r0], grow.at[par, 0], semg)
            pltpu.async_copy(gmax_hbm.at[r0 + 1], grow.at[par, 1], semg)

        def drain_gmax(par):
            for s in range(2):
                pltpu.make_async_copy(gmax_hbm.at[base], grow.at[par, s],
                                      semg).wait()

        def build_cand(par, p):
            gids = []
            for s in range(2):
                _, gid = tourney([(grow[par, s, pl.ds(c * GW, GW)],
                                   iota + c * GW) for c in range(NG // GW)])
                gids.append(gid)
                rowflat = (base + 2 * p + s) * N
                for c in range(KNN):
                    cidx[par, 2 * s + c // 8, pl.ds((c % 8) * GW, GW)] = (
                        rowflat + gid[c] + NG * iota)
            return gids

        def fire_cand(par):
            for j in range(4):
                pltpu.async_copy(scores_hbm.at[cidx.at[par, j]],
                                 cand.at[par, pl.ds(j * 128, 128)], semc)

        def drain_cand(par):
            for j in range(4):
                pltpu.make_async_copy(scores_hbm.at[pl.ds(0, 128)],
                                      cand.at[par, pl.ds(j * 128, 128)],
                                      semc).wait()

        def stage2(par, s, row, gid):
            _, jid = tourney([(cand[par, pl.ds((s * KNN + c) * GW, GW)],
                               gid[c] + NG * iota) for c in range(KNN)])
            rowbase = (row // N) * N
            return pltpu.async_copy(g_hbm.at[jid + rowbase], gbuf.at[s], seme)

        def reduce_store(s, row):
            for ch in range(C // GW):
                sl0 = ch * GW
                v = gbuf[s, 0, pl.ds(sl0, GW)]
                mx = v
                mn = v
                s1 = v
                s2 = v * v
                for rr in range(1, KNN):
                    v = gbuf[s, rr, pl.ds(sl0, GW)]
                    mx = jnp.maximum(mx, v)
                    mn = jnp.minimum(mn, v)
                    s1 = s1 + v
                    s2 = s2 + v * v
                orow[s, pl.ds(sl0, GW)] = mx
                orow[s, pl.ds(C + sl0, GW)] = mn
                orow[s, pl.ds(2 * C + sl0, GW)] = s1
                orow[s, pl.ds(3 * C + sl0, GW)] = s2
            return pltpu.async_copy(orow.at[s], out_hbm.at[row], semo)

        def step(r, par0, par1, g0, g1):
            @pl.when(r + 2 < npairs)
            def _():
                fire_gmax(r + 2, par0)

            drain_cand(par0)
            row0 = base + 2 * r
            e0 = stage2(par0, 0, row0, g0)
            e1 = stage2(par0, 1, row0 + 1, g1)

            @pl.when(r + 1 < npairs)
            def _():
                drain_gmax(par1)
            ng = build_cand(par1, r + 1)

            @pl.when(r + 1 < npairs)
            def _():
                fire_cand(par1)

            e0.wait()
            e1.wait()
            o0 = reduce_store(0, row0)
            o1 = reduce_store(1, row0 + 1)
            o0.wait()
            o1.wait()
            return ng

        def body(q, carry):
            g0, g1 = carry
            ng = step(2 * q, 0, 1, g0, g1)
            ng = step(2 * q + 1, 1, 0, ng[0], ng[1])
            return (ng[0], ng[1])

        fire_gmax(0, 0)
        fire_gmax(1, 1)
        drain_gmax(0)
        g_init = build_cand(0, 0)
        fire_cand(0)
        lax.fori_loop(0, npairs // 2, body, (g_init[0], g_init[1]))

    return sc_topk


def kernel(xyz_s, xyz_t, feature_s, feature_t, W, gamma, beta):
    B, C, N = feature_s.shape
    NG = N // GW
    NB = N // RB
    f32 = jnp.float32

    W1 = W[:, :C]
    W2 = W[:, C:2 * C]
    W3 = W[:, 2 * C:2 * C + 3]
    W4 = W[:, 2 * C + 3:]
    A = W1 + W2
    W1n = -W1
    W43 = W4 - W3
    gamma2 = gamma.reshape(1, C)
    beta2 = beta.reshape(1, C)

    blk_cr = pl.BlockSpec((1, C, RB), lambda b, i: (b, 0, i))
    blk_cn = pl.BlockSpec((1, C, N), lambda b, i: (b, 0, 0))
    blk_r3 = pl.BlockSpec((1, RB, 3), lambda b, i: (b, i, 0))
    blk_r64 = pl.BlockSpec((1, RB, C), lambda b, i: (b, i, 0))
    full2 = lambda shape: pl.BlockSpec(shape, lambda b, i: (0, 0))

    scores, gmax, g = pl.pallas_call(
        _scores_body,
        grid=(B, NB),
        in_specs=[blk_cr, blk_cn, blk_cn, blk_cr, blk_r3,
                  full2((C, C)), full2((C, 3))],
        out_specs=[pl.BlockSpec((1, RB, N), lambda b, i: (b, i, 0)),
                   pl.BlockSpec((1, RB, NG), lambda b, i: (b, i, 0)),
                   blk_r64],
        out_shape=[jax.ShapeDtypeStruct((B, N, N), f32),
                   jax.ShapeDtypeStruct((B, N, NG), f32),
                   jax.ShapeDtypeStruct((B, N, C), f32)],
    )(feature_s, feature_t, feature_s, feature_t, xyz_s, A, W3)

    sc_topk = _make_sc_topk(B, N, C, NG)
    out4 = sc_topk(
        gmax.reshape(B * N, NG),
        scores.reshape(B * N * N),
        g.reshape(B * N, C)).reshape(B, N, 4 * C)

    blk_o4 = pl.BlockSpec((1, RB, 4 * C), lambda b, i: (b, i, 0))
    stat_spec = pl.BlockSpec((8, C), lambda b, i: (0, 0))
    sumh, sqh = pl.pallas_call(
        _stats_body,
        grid=(B, NB),
        in_specs=[blk_o4, blk_cr, blk_r3,
                  full2((C, C)), full2((C, 3))],
        out_specs=[stat_spec, stat_spec],
        out_shape=[jax.ShapeDtypeStruct((8, C), f32),
                   jax.ShapeDtypeStruct((8, C), f32)],
        compiler_params=pltpu.CompilerParams(
            dimension_semantics=("arbitrary", "arbitrary")),
    )(out4, feature_t, xyz_t, W1n, W43)

    m_total = float(B * N * KNN)
    out = pl.pallas_call(
        functools.partial(_final_body, m_total=m_total),
        grid=(B, NB),
        in_specs=[blk_o4, blk_cr, blk_r3,
                  full2((C, C)), full2((C, 3)),
                  full2((1, C)), full2((1, C)),
                  stat_spec, stat_spec],
        out_specs=pl.BlockSpec((1, C, RB), lambda b, i: (b, 0, i)),
        out_shape=jax.ShapeDtypeStruct((B, C, N), f32),
    )(out4, feature_t, xyz_t, W1n, W43, gamma2, beta2, sumh, sqh)
    return out
